# baseline (speedup 1.0000x reference)
import functools

import jax
import jax.numpy as jnp
from jax import lax
from jax.experimental import pallas as pl
from jax.experimental.pallas import tpu as pltpu
from jax.experimental.pallas import tpu_sc as plsc

N = 10000
E = 320000
D = 128
B = 16
DH = D // 2
NW = 32
EPT = E // 16
CK = 80
NCH = EPT // CK
RPT = N // 16
SR = 125

_mesh = plsc.VectorSubcoreMesh(core_axis_name="c", subcore_axis_name="s")


@functools.partial(
    pl.kernel,
    out_type=jax.ShapeDtypeStruct((N, D), jnp.float32),
    mesh=_mesh,
    scratch_types=[
        pltpu.VMEM((CK,), jnp.int32),
        pltpu.VMEM((CK, D), jnp.float32),
        pltpu.SemaphoreType.DMA,
    ],
    compiler_params=pltpu.CompilerParams(use_tc_tiling_on_sc=False, needs_layout_passes=False),
)
def _emb_gather(x_hbm, emb_hbm, h_hbm, idx_v, rows_v, sem):
    cid = lax.axis_index("c")
    sid = lax.axis_index("s")
    wid = cid * 16 + sid
    nch = jnp.where(wid == NW - 1, 1, 4)

    def body(j, _):
        base = wid * 320 + j * CK
        pltpu.sync_copy(x_hbm.at[pl.ds(base, CK)], idx_v)
        pltpu.async_copy(emb_hbm.at[idx_v], rows_v, sem).wait()
        pltpu.sync_copy(rows_v, h_hbm.at[pl.ds(base, CK)])
        return 0

    lax.fori_loop(0, nch, body, 0)


@functools.partial(
    pl.kernel,
    out_type=jax.ShapeDtypeStruct((2, N, DH + 16), jnp.float32),
    mesh=_mesh,
    scratch_types=[
        pltpu.VMEM((N,), jnp.float32),
        pltpu.VMEM((N,), jnp.float32),
        [pltpu.VMEM((3 * CK,), jnp.int32)] * 4,
        [pltpu.VMEM((CK,), jnp.int32)] * 4,
        [pltpu.VMEM((1, CK), jnp.int32)] * 4,
        [pltpu.VMEM((CK, DH), jnp.float32)] * 4,
        [pltpu.VMEM((CK, DH + 16), jnp.float32)] * 4,
        pltpu.VMEM((CK,), jnp.float32),
        pltpu.VMEM((SR, DH + 16), jnp.float32),
        pltpu.VMEM_SHARED((N, DH + 16), jnp.float32),
        [pltpu.SemaphoreType.DMA] * 4,
        [pltpu.SemaphoreType.DMA] * 4,
        [pltpu.SemaphoreType.DMA] * 4,
    ],
    compiler_params=pltpu.CompilerParams(use_tc_tiling_on_sc=False, needs_layout_passes=False),
)
def _edge_pass(sde_hbm, ss_hbm, sd_hbm, h2x_hbm, acc_hbm,
               ss_v, sd_v, sdeb, gidx, dstb, rowsg, rowss, exb,
               stage, accs, lsem, gsem, ssem):
    cid = lax.axis_index("c")
    sid = lax.axis_index("s")
    zeros16 = jnp.zeros((16,), jnp.float32)
    DB = DH + 16

    pltpu.sync_copy(ss_hbm, ss_v)
    pltpu.sync_copy(sd_hbm, sd_v)

    def zb(i, _):
        r = i // (DB // 16)
        c = i % (DB // 16)
        stage[r, pl.ds(c * 16, 16)] = zeros16
        return 0
    lax.fori_loop(0, SR * (DB // 16), zb, 0)
    for t in range(RPT // SR):
        pltpu.sync_copy(stage, accs.at[pl.ds(sid * RPT + t * SR, SR)])
    plsc.subcore_barrier()

    lanes = lax.iota(jnp.int32, 16)
    lane0f = jnp.where(lanes == 0, 1.0, 0.0).astype(jnp.float32)
    lanes3 = lanes * 3

    def fire_sde(j, b):
        base = (sid * EPT + j * CK) * 3
        pltpu.async_copy(sde_hbm.at[pl.ds(base, 3 * CK)], sdeb[b], lsem[b])

    def wait_sde(j, b):
        base = (sid * EPT + j * CK) * 3
        pltpu.make_async_copy(sde_hbm.at[pl.ds(base, 3 * CK)], sdeb[b],
                              lsem[b]).wait()

    def fire_gather(j, b):
        wait_sde(j, b)
        for v in range(CK // 16):
            sl = pl.ds(v * 16, 16)
            sv = plsc.load_gather(sdeb[b], [lanes3 + (3 * 16 * v)])
            gidx[b][sl] = sv * 2 + cid
        pltpu.async_copy(h2x_hbm.at[gidx[b]], rowsg[b], gsem[b])

    def wait_scatter(b):
        pltpu.make_async_copy(rowss[b], accs.at[dstb[b].at[0]], ssem[b]).wait()

    def process(j, b):
        for v in range(CK // 16):
            sl = pl.ds(v * 16, 16)
            dv = plsc.load_gather(sdeb[b], [lanes3 + (3 * 16 * v + 1)])
            ev = plsc.bitcast(
                plsc.load_gather(sdeb[b], [lanes3 + (3 * 16 * v + 2)]),
                jnp.float32)
            dstb[b][0, sl] = dv
            sv = lax.shift_right_logical(gidx[b][sl] - cid, 1)
            a = plsc.load_gather(sd_v, [dv]) + plsc.load_gather(ss_v, [sv])
            a = jnp.where(a >= 0, a, 0.2 * a) * ev
            exb[sl] = jnp.exp(a)

        pltpu.make_async_copy(h2x_hbm.at[gidx[b]], rowsg[b], gsem[b]).wait()

        def rowgrp(g, _):
            r0 = g * 16
            exv16 = exb[pl.ds(r0, 16)]
            for u in range(16):
                exv = jnp.full((16,), exv16[u], jnp.float32)
                for c in range(DH // 16):
                    sl = pl.ds(c * 16, 16)
                    rowss[b][r0 + u, sl] = rowsg[b][r0 + u, sl] * exv
                rowss[b][r0 + u, pl.ds(DH, 16)] = exv * lane0f
            return 0
        lax.fori_loop(0, CK // 16, rowgrp, 0)

        pltpu.async_copy(rowss[b], accs.at[dstb[b].at[0]], ssem[b], add=True)

    for b in range(4):
        fire_sde(b, b)
    fire_gather(0, 0)
    fire_gather(1, 1)

    def quadbody(q, _):
        for u in range(4):
            j = 4 * q + u
            b2 = (u + 2) % 4

            @pl.when(j >= 2)
            def _():
                wait_scatter(b2)
            fire_gather(j + 2, b2)
            process(j, u)

            @pl.when(j + 4 < NCH)
            def _():
                fire_sde(j + 4, u)
        return 0

    lax.fori_loop(0, (NCH - 2) // 4, quadbody, 0)

    wait_scatter(2)
    process(NCH - 2, 0)
    wait_scatter(3)
    process(NCH - 1, 1)
    wait_scatter(0)
    wait_scatter(1)
    plsc.subcore_barrier()

    for t in range(RPT // SR):
        rb = sid * RPT + t * SR
        pltpu.sync_copy(accs.at[pl.ds(rb, SR)], stage)
        pltpu.sync_copy(stage, acc_hbm.at[cid, pl.ds(rb, SR)])


def _tc_first_body(h_ref, w_ref, asc_ref, adc_ref, H_ref, ss_ref, sd_ref, exs_ref):
    H = lax.dot_general(h_ref[...], w_ref[...], (((1,), (1,)), ((), ())),
                        preferred_element_type=jnp.float32)
    H_ref[...] = H
    ss = lax.dot_general(H, asc_ref[...], (((1,), (0,)), ((), ())),
                         preferred_element_type=jnp.float32)
    sd = lax.dot_general(H, adc_ref[...], (((1,), (0,)), ((), ())),
                         preferred_element_type=jnp.float32)
    ss_ref[...] = ss
    sd_ref[...] = sd
    a0 = ss + sd
    exs_ref[...] = jnp.exp(jnp.where(a0 >= 0, a0, 0.2 * a0))


def _tc_first(h, W, asc, adc):
    return pl.pallas_call(
        _tc_first_body,
        out_shape=(
            jax.ShapeDtypeStruct((N, D), jnp.float32),
            jax.ShapeDtypeStruct((N, 1), jnp.float32),
            jax.ShapeDtypeStruct((N, 1), jnp.float32),
            jax.ShapeDtypeStruct((N, 1), jnp.float32),
        ),
    )(h, W, asc, adc)


def _combine(acc_ref, H_ref, exs_ref):
    num = jnp.concatenate([acc_ref[0, :, :DH], acc_ref[1, :, :DH]], axis=1)
    num = num + exs_ref[...] * H_ref[...]
    den = acc_ref[0, :, DH:DH + 1] + exs_ref[...] + 1e-16
    return jnp.maximum(num / den, 0.0)


def _tc_mid_body(acc_ref, H_ref, exs_ref, w_ref, asc_ref, adc_ref,
                 H2_ref, ss_ref, sd_ref, exs2_ref):
    h1 = _combine(acc_ref, H_ref, exs_ref)
    H2 = lax.dot_general(h1, w_ref[...], (((1,), (1,)), ((), ())),
                         preferred_element_type=jnp.float32)
    H2_ref[...] = H2
    ss = lax.dot_general(H2, asc_ref[...], (((1,), (0,)), ((), ())),
                         preferred_element_type=jnp.float32)
    sd = lax.dot_general(H2, adc_ref[...], (((1,), (0,)), ((), ())),
                         preferred_element_type=jnp.float32)
    ss_ref[...] = ss
    sd_ref[...] = sd
    a0 = ss + sd
    exs2_ref[...] = jnp.exp(jnp.where(a0 >= 0, a0, 0.2 * a0))


def _tc_mid(acc, H, exs, W, asc, adc):
    return pl.pallas_call(
        _tc_mid_body,
        out_shape=(
            jax.ShapeDtypeStruct((N, D), jnp.float32),
            jax.ShapeDtypeStruct((N, 1), jnp.float32),
            jax.ShapeDtypeStruct((N, 1), jnp.float32),
            jax.ShapeDtypeStruct((N, 1), jnp.float32),
        ),
    )(acc, H, exs, W, asc, adc)


def _tc_final_body(acc_ref, H_ref, exs_ref, bt_ref, wm1_ref, bm1_ref,
                   wm2p_ref, bm2p_ref, out_ref):
    h2 = _combine(acc_ref, H_ref, exs_ref)
    biota = lax.broadcasted_iota(jnp.int32, (B, N), 0)
    Mf = jnp.where(biota == bt_ref[...], 1.0, 0.0)
    g = lax.dot_general(Mf, h2, (((1,), (0,)), ((), ())),
                        preferred_element_type=jnp.float32)
    cnt = jnp.sum(Mf, axis=1, keepdims=True)
    g = g / jnp.maximum(cnt, 1.0)
    z = lax.dot_general(g, wm1_ref[...], (((1,), (1,)), ((), ())),
                        preferred_element_type=jnp.float32) + bm1_ref[...]
    z = jnp.maximum(z, 0.0)
    out_ref[...] = lax.dot_general(z, wm2p_ref[...], (((1,), (1,)), ((), ())),
                                   preferred_element_type=jnp.float32) + bm2p_ref[...]


def _tc_final(acc, H, exs, bt, wm1, bm1, wm2p, bm2p):
    return pl.pallas_call(
        _tc_final_body,
        out_shape=jax.ShapeDtypeStruct((B, D), jnp.float32),
    )(acc, H, exs, bt, wm1, bm1, wm2p, bm2p)


@jax.jit
def kernel(x, edge_index, edge_attr, batch, emb, W1, as1, ad1, W2, as2, ad2,
           Wm1, bm1, Wm2, bm2):
    src = edge_index[0].astype(jnp.int32)
    dst = edge_index[1].astype(jnp.int32)
    x32 = x.astype(jnp.int32)
    ea = edge_attr.astype(jnp.float32)

    h = _emb_gather(x32, emb)

    H1, ss1, sd1, exs1 = _tc_first(h, W1, as1.reshape(D, 1), ad1.reshape(D, 1))
    sde = jnp.stack(
        [src, dst, lax.bitcast_convert_type(ea, jnp.int32)], axis=1
    ).reshape(3 * E)
    acc1 = _edge_pass(sde, ss1.reshape(N), sd1.reshape(N),
                      H1.reshape(2 * N, DH))

    H2, ss2, sd2, exs2 = _tc_mid(acc1, H1, exs1, W2,
                                 as2.reshape(D, 1), ad2.reshape(D, 1))
    acc2 = _edge_pass(sde, ss2.reshape(N), sd2.reshape(N),
                      H2.reshape(2 * N, DH))

    wm2p = jnp.zeros((D, D // 2), jnp.float32).at[:2].set(Wm2)
    bm2p = jnp.zeros((1, D), jnp.float32).at[0, :2].set(bm2)
    out = _tc_final(acc2, H2, exs2, batch.astype(jnp.int32).reshape(1, N),
                    Wm1, bm1.reshape(1, D // 2), wm2p, bm2p)
    return out[:, :2]


# speedup vs baseline: 1.0044x; 1.0044x over previous
import functools

import jax
import jax.numpy as jnp
from jax import lax
from jax.experimental import pallas as pl
from jax.experimental.pallas import tpu as pltpu
from jax.experimental.pallas import tpu_sc as plsc

N = 10000
E = 320000
D = 128
B = 16
DH = D // 2
NW = 32
EPT = E // 16
CK = 80
NCH = EPT // CK
RPT = N // 16
SR = 125

_mesh = plsc.VectorSubcoreMesh(core_axis_name="c", subcore_axis_name="s")


@functools.partial(
    pl.kernel,
    out_type=jax.ShapeDtypeStruct((N, D), jnp.float32),
    mesh=_mesh,
    scratch_types=[
        pltpu.VMEM((CK,), jnp.int32),
        pltpu.VMEM((CK, D), jnp.float32),
        pltpu.SemaphoreType.DMA,
    ],
    compiler_params=pltpu.CompilerParams(use_tc_tiling_on_sc=False, needs_layout_passes=False),
)
def _emb_gather(x_hbm, emb_hbm, h_hbm, idx_v, rows_v, sem):
    cid = lax.axis_index("c")
    sid = lax.axis_index("s")
    wid = cid * 16 + sid
    nch = jnp.where(wid == NW - 1, 1, 4)

    def body(j, _):
        base = wid * 320 + j * CK
        pltpu.sync_copy(x_hbm.at[pl.ds(base, CK)], idx_v)
        pltpu.async_copy(emb_hbm.at[idx_v], rows_v, sem).wait()
        pltpu.sync_copy(rows_v, h_hbm.at[pl.ds(base, CK)])
        return 0

    lax.fori_loop(0, nch, body, 0)


@functools.partial(
    pl.kernel,
    out_type=jax.ShapeDtypeStruct((2, N, DH + 16), jnp.float32),
    mesh=_mesh,
    scratch_types=[
        pltpu.VMEM((N,), jnp.float32),
        pltpu.VMEM((N,), jnp.float32),
        [pltpu.VMEM((3 * CK,), jnp.int32)] * 4,
        [pltpu.VMEM((CK,), jnp.int32)] * 4,
        [pltpu.VMEM((1, CK), jnp.int32)] * 4,
        [pltpu.VMEM((CK, DH), jnp.float32)] * 4,
        [pltpu.VMEM((CK, DH + 16), jnp.float32)] * 4,
        pltpu.VMEM((CK,), jnp.float32),
        pltpu.VMEM((SR, DH + 16), jnp.float32),
        pltpu.VMEM_SHARED((N, DH + 16), jnp.float32),
        [pltpu.SemaphoreType.DMA] * 4,
        [pltpu.SemaphoreType.DMA] * 4,
        [pltpu.SemaphoreType.DMA] * 4,
    ],
    compiler_params=pltpu.CompilerParams(use_tc_tiling_on_sc=False, needs_layout_passes=False),
)
def _edge_pass(sde_hbm, ss_hbm, sd_hbm, h2x_hbm, acc_hbm,
               ss_v, sd_v, sdeb, gidx, dstb, rowsg, rowss, exb,
               stage, accs, lsem, gsem, ssem):
    cid = lax.axis_index("c")
    sid = lax.axis_index("s")
    zeros16 = jnp.zeros((16,), jnp.float32)
    DB = DH + 16

    pltpu.sync_copy(ss_hbm, ss_v)
    pltpu.sync_copy(sd_hbm, sd_v)

    def zb(i, _):
        r = i // (DB // 16)
        c = i % (DB // 16)
        stage[r, pl.ds(c * 16, 16)] = zeros16
        return 0
    lax.fori_loop(0, SR * (DB // 16), zb, 0)
    for t in range(RPT // SR):
        pltpu.sync_copy(stage, accs.at[pl.ds(sid * RPT + t * SR, SR)])
    plsc.subcore_barrier()

    lanes = lax.iota(jnp.int32, 16)
    lane0f = jnp.where(lanes == 0, 1.0, 0.0).astype(jnp.float32)
    lanes3 = lanes * 3

    def fire_sde(j, b):
        base = (sid * EPT + j * CK) * 3
        pltpu.async_copy(sde_hbm.at[pl.ds(base, 3 * CK)], sdeb[b], lsem[b])

    def wait_sde(j, b):
        base = (sid * EPT + j * CK) * 3
        pltpu.make_async_copy(sde_hbm.at[pl.ds(base, 3 * CK)], sdeb[b],
                              lsem[b]).wait()

    def fire_gather(j, b):
        wait_sde(j, b)
        for v in range(CK // 16):
            sl = pl.ds(v * 16, 16)
            sv = plsc.load_gather(sdeb[b], [lanes3 + (3 * 16 * v)])
            gidx[b][sl] = sv + cid * N
        pltpu.async_copy(h2x_hbm.at[gidx[b]], rowsg[b], gsem[b])

    def wait_scatter(b):
        pltpu.make_async_copy(rowss[b], accs.at[dstb[b].at[0]], ssem[b]).wait()

    def process(j, b):
        for v in range(CK // 16):
            sl = pl.ds(v * 16, 16)
            dv = plsc.load_gather(sdeb[b], [lanes3 + (3 * 16 * v + 1)])
            ev = plsc.bitcast(
                plsc.load_gather(sdeb[b], [lanes3 + (3 * 16 * v + 2)]),
                jnp.float32)
            dstb[b][0, sl] = dv
            sv = gidx[b][sl] - cid * N
            a = plsc.load_gather(sd_v, [dv]) + plsc.load_gather(ss_v, [sv])
            a = jnp.where(a >= 0, a, 0.2 * a) * ev
            exb[sl] = jnp.exp(a)

        pltpu.make_async_copy(h2x_hbm.at[gidx[b]], rowsg[b], gsem[b]).wait()

        def rowgrp(g, _):
            r0 = g * 16
            exv16 = exb[pl.ds(r0, 16)]
            for u in range(16):
                exv = jnp.full((16,), exv16[u], jnp.float32)
                for c in range(DH // 16):
                    sl = pl.ds(c * 16, 16)
                    rowss[b][r0 + u, sl] = rowsg[b][r0 + u, sl] * exv
                rowss[b][r0 + u, pl.ds(DH, 16)] = exv * lane0f
            return 0
        lax.fori_loop(0, CK // 16, rowgrp, 0)

        pltpu.async_copy(rowss[b], accs.at[dstb[b].at[0]], ssem[b], add=True)

    for b in range(4):
        fire_sde(b, b)
    fire_gather(0, 0)
    fire_gather(1, 1)

    def quadbody(q, _):
        for u in range(4):
            j = 4 * q + u
            b2 = (u + 2) % 4

            @pl.when(j >= 2)
            def _():
                wait_scatter(b2)
            fire_gather(j + 2, b2)
            process(j, u)

            @pl.when(j + 4 < NCH)
            def _():
                fire_sde(j + 4, u)
        return 0

    lax.fori_loop(0, (NCH - 2) // 4, quadbody, 0)

    wait_scatter(2)
    process(NCH - 2, 0)
    wait_scatter(3)
    process(NCH - 1, 1)
    wait_scatter(0)
    wait_scatter(1)
    plsc.subcore_barrier()

    for t in range(RPT // SR):
        rb = sid * RPT + t * SR
        pltpu.sync_copy(accs.at[pl.ds(rb, SR)], stage)
        pltpu.sync_copy(stage, acc_hbm.at[cid, pl.ds(rb, SR)])


def _tc_first_body(h_ref, w_ref, asc_ref, adc_ref, H_ref, ss_ref, sd_ref, exs_ref):
    H = lax.dot_general(h_ref[...], w_ref[...], (((1,), (1,)), ((), ())),
                        preferred_element_type=jnp.float32)
    H_ref[0] = H[:, :DH]
    H_ref[1] = H[:, DH:]
    ss = lax.dot_general(H, asc_ref[...], (((1,), (0,)), ((), ())),
                         preferred_element_type=jnp.float32)
    sd = lax.dot_general(H, adc_ref[...], (((1,), (0,)), ((), ())),
                         preferred_element_type=jnp.float32)
    ss_ref[...] = ss
    sd_ref[...] = sd
    a0 = ss + sd
    exs_ref[...] = jnp.exp(jnp.where(a0 >= 0, a0, 0.2 * a0))


_NB = 5
_RB = N // _NB


def _tc_first(h, W, asc, adc):
    return pl.pallas_call(
        _tc_first_body,
        grid=(_NB,),
        in_specs=[
            pl.BlockSpec((_RB, D), lambda i: (i, 0)),
            pl.BlockSpec((D, D), lambda i: (0, 0)),
            pl.BlockSpec((D, 1), lambda i: (0, 0)),
            pl.BlockSpec((D, 1), lambda i: (0, 0)),
        ],
        out_specs=(
            pl.BlockSpec((2, _RB, DH), lambda i: (0, i, 0)),
            pl.BlockSpec((_RB, 1), lambda i: (i, 0)),
            pl.BlockSpec((_RB, 1), lambda i: (i, 0)),
            pl.BlockSpec((_RB, 1), lambda i: (i, 0)),
        ),
        out_shape=(
            jax.ShapeDtypeStruct((2, N, DH), jnp.float32),
            jax.ShapeDtypeStruct((N, 1), jnp.float32),
            jax.ShapeDtypeStruct((N, 1), jnp.float32),
            jax.ShapeDtypeStruct((N, 1), jnp.float32),
        ),
    )(h, W, asc, adc)


def _combine(acc_ref, H_ref, exs_ref):
    H = jnp.concatenate([H_ref[0], H_ref[1]], axis=1)
    num = jnp.concatenate([acc_ref[0, :, :DH], acc_ref[1, :, :DH]], axis=1)
    num = num + exs_ref[...] * H
    den = acc_ref[0, :, DH:DH + 1] + exs_ref[...] + 1e-16
    return jnp.maximum(num / den, 0.0)


def _tc_mid_body(acc_ref, H_ref, exs_ref, w_ref, asc_ref, adc_ref,
                 H2_ref, ss_ref, sd_ref, exs2_ref):
    h1 = _combine(acc_ref, H_ref, exs_ref)
    H2 = lax.dot_general(h1, w_ref[...], (((1,), (1,)), ((), ())),
                         preferred_element_type=jnp.float32)
    H2_ref[0] = H2[:, :DH]
    H2_ref[1] = H2[:, DH:]
    ss = lax.dot_general(H2, asc_ref[...], (((1,), (0,)), ((), ())),
                         preferred_element_type=jnp.float32)
    sd = lax.dot_general(H2, adc_ref[...], (((1,), (0,)), ((), ())),
                         preferred_element_type=jnp.float32)
    ss_ref[...] = ss
    sd_ref[...] = sd
    a0 = ss + sd
    exs2_ref[...] = jnp.exp(jnp.where(a0 >= 0, a0, 0.2 * a0))


def _tc_mid(acc, H, exs, W, asc, adc):
    return pl.pallas_call(
        _tc_mid_body,
        grid=(_NB,),
        in_specs=[
            pl.BlockSpec((2, _RB, DH + 16), lambda i: (0, i, 0)),
            pl.BlockSpec((2, _RB, DH), lambda i: (0, i, 0)),
            pl.BlockSpec((_RB, 1), lambda i: (i, 0)),
            pl.BlockSpec((D, D), lambda i: (0, 0)),
            pl.BlockSpec((D, 1), lambda i: (0, 0)),
            pl.BlockSpec((D, 1), lambda i: (0, 0)),
        ],
        out_specs=(
            pl.BlockSpec((2, _RB, DH), lambda i: (0, i, 0)),
            pl.BlockSpec((_RB, 1), lambda i: (i, 0)),
            pl.BlockSpec((_RB, 1), lambda i: (i, 0)),
            pl.BlockSpec((_RB, 1), lambda i: (i, 0)),
        ),
        out_shape=(
            jax.ShapeDtypeStruct((2, N, DH), jnp.float32),
            jax.ShapeDtypeStruct((N, 1), jnp.float32),
            jax.ShapeDtypeStruct((N, 1), jnp.float32),
            jax.ShapeDtypeStruct((N, 1), jnp.float32),
        ),
    )(acc, H, exs, W, asc, adc)


def _tc_final_body(acc_ref, H_ref, exs_ref, bt_ref, wm1_ref, bm1_ref,
                   wm2p_ref, bm2p_ref, out_ref):
    h2 = _combine(acc_ref, H_ref, exs_ref)
    biota = lax.broadcasted_iota(jnp.int32, (B, N), 0)
    Mf = jnp.where(biota == bt_ref[...], 1.0, 0.0)
    g = lax.dot_general(Mf, h2, (((1,), (0,)), ((), ())),
                        preferred_element_type=jnp.float32)
    cnt = jnp.sum(Mf, axis=1, keepdims=True)
    g = g / jnp.maximum(cnt, 1.0)
    z = lax.dot_general(g, wm1_ref[...], (((1,), (1,)), ((), ())),
                        preferred_element_type=jnp.float32) + bm1_ref[...]
    z = jnp.maximum(z, 0.0)
    out_ref[...] = lax.dot_general(z, wm2p_ref[...], (((1,), (1,)), ((), ())),
                                   preferred_element_type=jnp.float32) + bm2p_ref[...]


def _tc_final(acc, H, exs, bt, wm1, bm1, wm2p, bm2p):
    return pl.pallas_call(
        _tc_final_body,
        out_shape=jax.ShapeDtypeStruct((B, D), jnp.float32),
    )(acc, H, exs, bt, wm1, bm1, wm2p, bm2p)


@jax.jit
def kernel(x, edge_index, edge_attr, batch, emb, W1, as1, ad1, W2, as2, ad2,
           Wm1, bm1, Wm2, bm2):
    src = edge_index[0].astype(jnp.int32)
    dst = edge_index[1].astype(jnp.int32)
    x32 = x.astype(jnp.int32)
    ea = edge_attr.astype(jnp.float32)

    h = _emb_gather(x32, emb)

    H1, ss1, sd1, exs1 = _tc_first(h, W1, as1.reshape(D, 1), ad1.reshape(D, 1))
    sde = jnp.stack(
        [src, dst, lax.bitcast_convert_type(ea, jnp.int32)], axis=1
    ).reshape(3 * E)
    acc1 = _edge_pass(sde, ss1.reshape(N), sd1.reshape(N),
                      H1.reshape(2 * N, DH))

    H2, ss2, sd2, exs2 = _tc_mid(acc1, H1, exs1, W2,
                                 as2.reshape(D, 1), ad2.reshape(D, 1))
    acc2 = _edge_pass(sde, ss2.reshape(N), sd2.reshape(N),
                      H2.reshape(2 * N, DH))

    wm2p = jnp.zeros((D, D // 2), jnp.float32).at[:2].set(Wm2)
    bm2p = jnp.zeros((1, D), jnp.float32).at[0, :2].set(bm2)
    out = _tc_final(acc2, H2, exs2, batch.astype(jnp.int32).reshape(1, N),
                    Wm1, bm1.reshape(1, D // 2), wm2p, bm2p)
    return out[:, :2]
